# baseline (speedup 1.0000x reference)
import functools

import jax
import jax.numpy as jnp
from jax import lax
from jax.experimental import pallas as pl
from jax.experimental.pallas import tpu as pltpu
from jax.experimental.pallas import tpu_sc as plsc

F32 = jnp.float32

_NC = 2
_NS = 16
_NW = _NC * _NS
_EB = 1024
_IB = 128
_TC_EDGE_BLK = 4096
_NODE_BLK = 2000
_G = 64


def _sc_gather(table, idx2d, n_rows, slow_frac_num=1, slow_frac_den=4):
    d = table.shape[1]
    part = 2560
    unit = _NS * part
    rows_c0 = max(unit, (n_rows * slow_frac_num // slow_frac_den)
                  // unit * unit)
    rows_c1 = n_rows - rows_c0
    mesh = plsc.VectorSubcoreMesh(core_axis_name="c", subcore_axis_name="s")

    max_ops = max(rows_c0, rows_c1) // _NS // _IB

    @functools.partial(
        pl.kernel,
        mesh=mesh,
        out_type=jax.ShapeDtypeStruct((n_rows, d), F32),
        scratch_types=[
            pltpu.VMEM((max_ops, _IB), jnp.int32),
            pltpu.VMEM((part, d), F32),
            pltpu.SemaphoreType.DMA,
        ],
        compiler_params=pltpu.CompilerParams(use_tc_tiling_on_sc=False),
    )
    def k(table_hbm, idx_hbm, out_hbm, idx_v, rows_v, sem):
        cid = lax.axis_index("c")
        sid = lax.axis_index("s")

        def run_core(core_base, rows_per_tile):
            tbase = core_base + sid * rows_per_tile
            n_ops_t = rows_per_tile // _IB
            pltpu.sync_copy(idx_hbm.at[pl.ds(tbase // _IB, n_ops_t)],
                            idx_v.at[pl.ds(0, n_ops_t)])
            for pt in range(rows_per_tile // part):
                copies = []
                for j in range(part // _IB):
                    jj = pt * (part // _IB) + j
                    copies.append(pltpu.async_copy(
                        table_hbm.at[idx_v.at[jj]],
                        rows_v.at[pl.ds(j * _IB, _IB)],
                        sem,
                    ))
                for c in copies:
                    c.wait()
                pltpu.sync_copy(
                    rows_v, out_hbm.at[pl.ds(tbase + pt * part, part)])

        @pl.when(cid == 0)
        def _():
            run_core(0, rows_c0 // _NS)

        @pl.when(cid == 1)
        def _():
            run_core(rows_c0, rows_c1 // _NS)

    return k(table, idx2d)


def _sc_scatter(msg, dst2d, zinit):
    n_rows, d = msg.shape
    n_pad = zinit.shape[0]
    rows_per_w = n_rows // _NW
    chunks = rows_per_w // _EB
    node_rows_per_tile = n_pad // _NS
    mesh = plsc.VectorSubcoreMesh(core_axis_name="c", subcore_axis_name="s")

    n_ops = rows_per_w // _IB
    half = rows_per_w // 2

    @functools.partial(
        pl.kernel,
        mesh=mesh,
        out_type=jax.ShapeDtypeStruct((_NC, n_pad, d), F32),
        scratch_types=[
            pltpu.VMEM((n_ops, _IB), jnp.int32),
            pltpu.VMEM((half, d), F32),
            pltpu.VMEM_SHARED((n_pad, d), F32),
            pltpu.SemaphoreType.DMA,
        ],
        compiler_params=pltpu.CompilerParams(use_tc_tiling_on_sc=False),
    )
    def k(msg_hbm, dst_hbm, z_hbm, out_hbm, idx_v, msg_v, shared, sem):
        cid = lax.axis_index("c")
        sid = lax.axis_index("s")
        wid = sid * _NC + cid
        tb = sid * node_rows_per_tile
        pltpu.sync_copy(z_hbm.at[pl.ds(tb, node_rows_per_tile)],
                        shared.at[pl.ds(tb, node_rows_per_tile)])
        pltpu.sync_copy(dst_hbm.at[pl.ds(wid * n_ops, n_ops)], idx_v)
        plsc.subcore_barrier()
        for hlf in range(2):
            pltpu.sync_copy(msg_hbm.at[pl.ds(wid * rows_per_w + hlf * half,
                                             half)], msg_v)
            copies = []
            for j in range(n_ops // 2):
                jj = hlf * (n_ops // 2) + j
                copies.append(pltpu.async_copy(
                    msg_v.at[pl.ds(j * _IB, _IB)],
                    shared.at[idx_v.at[jj]],
                    sem,
                    add=True,
                ))
            for c in copies:
                c.wait()
        plsc.subcore_barrier()
        pltpu.sync_copy(shared.at[pl.ds(tb, node_rows_per_tile)],
                        out_hbm.at[cid, pl.ds(tb, node_rows_per_tile)])

    return k(msg, dst2d, zinit)


def _tc_msg(xj, ea, wcatb, rmat, ed, h):
    n_rows = xj.shape[0]
    kw = ed * h

    def body(xj_ref, ea_ref, w_ref, r_ref, out_ref):
        p = jnp.dot(xj_ref[...], w_ref[...], preferred_element_type=F32)
        er = jnp.dot(ea_ref[...], r_ref[...], preferred_element_type=F32)
        t = er * p[:, :kw]
        u = t[:, 0:128]
        for i in range(1, kw // 128):
            u = u + t[:, i * 128:(i + 1) * 128]
        m = p[:, kw:]
        for j in range(128 // h):
            m = m + u[:, j * h:(j + 1) * h]
        out_ref[...] = m

    grid = n_rows // _TC_EDGE_BLK
    return pl.pallas_call(
        body,
        grid=(grid,),
        in_specs=[
            pl.BlockSpec((_TC_EDGE_BLK, xj.shape[1]), lambda i: (i, 0)),
            pl.BlockSpec((_TC_EDGE_BLK, ea.shape[1]), lambda i: (i, 0)),
            pl.BlockSpec((wcatb.shape[0], kw + h), lambda i: (0, 0)),
            pl.BlockSpec((ed, kw), lambda i: (0, 0)),
        ],
        out_specs=pl.BlockSpec((_TC_EDGE_BLK, h), lambda i: (i, 0)),
        out_shape=jax.ShapeDtypeStruct((n_rows, h), F32),
    )(xj, ea, wcatb, rmat)


def _tc_combine(p, x, root, bias):
    n, d = x.shape
    h = root.shape[1]

    def body(p_ref, x_ref, r_ref, b_ref, out_ref):
        agg = p_ref[0] + p_ref[1]
        xr = jnp.dot(x_ref[...], r_ref[...], preferred_element_type=F32)
        out_ref[...] = jnp.maximum(agg + xr + b_ref[...], 0.0)

    grid = n // _NODE_BLK
    return pl.pallas_call(
        body,
        grid=(grid,),
        in_specs=[
            pl.BlockSpec((2, _NODE_BLK, h), lambda i: (0, i, 0)),
            pl.BlockSpec((_NODE_BLK, d), lambda i: (i, 0)),
            pl.BlockSpec((d, h), lambda i: (0, 0)),
            pl.BlockSpec((1, h), lambda i: (0, 0)),
        ],
        out_specs=pl.BlockSpec((_NODE_BLK, h), lambda i: (i, 0)),
        out_shape=jax.ShapeDtypeStruct((n, h), F32),
    )(p, x, root, bias)


def _tc_final(p, h1, root2, bias2, batch3, wp1, bp1, wp2, bp2):
    n, h = h1.shape
    out_d = wp2.shape[1]
    grid = n // _NODE_BLK

    def body(p_ref, h1_ref, r2_ref, b2_ref, b_ref, wp1_ref, bp1_ref,
             wp2_ref, bp2_ref, out_ref, pool_acc, cnt_acc):
        i = pl.program_id(0)

        @pl.when(i == 0)
        def _():
            pool_acc[...] = jnp.zeros_like(pool_acc)
            cnt_acc[...] = jnp.zeros_like(cnt_acc)

        hr = jnp.dot(h1_ref[...], r2_ref[...], preferred_element_type=F32)
        h2 = jnp.maximum(p_ref[0] + p_ref[1] + hr + b2_ref[...], 0.0)
        b = b_ref[0, 0, :]
        onehot = (b[:, None] == lax.broadcasted_iota(jnp.int32, (1, _G), 1)
                  ).astype(F32)
        pool_acc[...] += lax.dot_general(
            onehot, h2, (((0,), (0,)), ((), ())), preferred_element_type=F32)
        cnt_acc[...] += lax.dot_general(
            onehot, jnp.ones((_NODE_BLK, 8), F32),
            (((0,), (0,)), ((), ())), preferred_element_type=F32)

        @pl.when(i == pl.num_programs(0) - 1)
        def _():
            cnt = jnp.maximum(cnt_acc[:, 0:1], 1.0)
            pooled = pool_acc[...] / cnt
            o = jnp.dot(pooled, wp1_ref[...],
                        preferred_element_type=F32) + bp1_ref[...]
            out_ref[...] = jnp.dot(o, wp2_ref[...],
                                   preferred_element_type=F32) + bp2_ref[...]

    return pl.pallas_call(
        body,
        grid=(grid,),
        in_specs=[
            pl.BlockSpec((2, _NODE_BLK, h), lambda i: (0, i, 0)),
            pl.BlockSpec((_NODE_BLK, h), lambda i: (i, 0)),
            pl.BlockSpec((h, h), lambda i: (0, 0)),
            pl.BlockSpec((1, h), lambda i: (0, 0)),
            pl.BlockSpec((1, 1, _NODE_BLK), lambda i: (i, 0, 0)),
            pl.BlockSpec((h, h), lambda i: (0, 0)),
            pl.BlockSpec((1, h), lambda i: (0, 0)),
            pl.BlockSpec((h, out_d), lambda i: (0, 0)),
            pl.BlockSpec((1, out_d), lambda i: (0, 0)),
        ],
        out_specs=pl.BlockSpec((_G, out_d), lambda i: (0, 0)),
        out_shape=jax.ShapeDtypeStruct((_G, out_d), F32),
        scratch_shapes=[
            pltpu.VMEM((_G, h), F32),
            pltpu.VMEM((_G, 8), F32),
        ],
    )(p, h1, root2, bias2, batch3, wp1, bp1, wp2, bp2)


def _fold_edge_weight(we, in_ch, ed, h):
    return we.reshape(ed, in_ch, h).transpose(1, 0, 2).reshape(in_ch, ed * h)


def kernel(x, edge_index, edge_attr, batch, W1e, b1e, root1, bias1,
           W2e, b2e, root2, bias2, Wp1, bp1, Wp2, bp2):
    n, in_ch = x.shape
    e = edge_index.shape[1]
    ed = edge_attr.shape[1]
    h = root1.shape[1]

    unit = _NW * _EB
    epad = ((e + unit - 1) // unit) * unit
    pad = epad - e
    src = jnp.concatenate([edge_index[0], jnp.zeros((pad,), jnp.int32)])
    dst = jnp.concatenate([edge_index[1], jnp.full((pad,), n, jnp.int32)])
    ea = jnp.concatenate([edge_attr, jnp.zeros((pad, ed), F32)], axis=0)
    src2d = src.reshape(-1, _IB)
    dst2d = dst.reshape(-1, _IB)

    n_pad = ((n + _IB * _NS) // (_IB * _NS)) * (_IB * _NS)
    zinit = jnp.zeros((n_pad, h), F32)

    kw = ed * h
    wc1 = jnp.concatenate(
        [_fold_edge_weight(W1e, in_ch, ed, h), b1e.reshape(in_ch, h)], axis=1)
    wc2 = jnp.concatenate(
        [_fold_edge_weight(W2e, h, ed, h), b2e.reshape(h, h)], axis=1)
    rmat = (jnp.arange(kw)[None, :] // h == jnp.arange(ed)[:, None]
            ).astype(F32)

    xj = _sc_gather(x, src2d, epad)
    msg1 = _tc_msg(xj, ea, wc1, rmat, ed, h)
    p1 = _sc_scatter(msg1, dst2d, zinit)
    h1 = _tc_combine(p1[:, :n, :], x, root1, bias1.reshape(1, h))

    hj = _sc_gather(h1, src2d, epad)
    msg2 = _tc_msg(hj, ea, wc2, rmat, ed, h)
    p2 = _sc_scatter(msg2, dst2d, zinit)

    batch3 = batch.reshape(n // _NODE_BLK, 1, _NODE_BLK)
    return _tc_final(p2[:, :n, :], h1, root2, bias2.reshape(1, h),
                     batch3, Wp1, bp1.reshape(1, h), Wp2,
                     bp2.reshape(1, Wp2.shape[1]))


# speedup vs baseline: 1.0113x; 1.0113x over previous
import functools

import jax
import jax.numpy as jnp
from jax import lax
from jax.experimental import pallas as pl
from jax.experimental.pallas import tpu as pltpu
from jax.experimental.pallas import tpu_sc as plsc

F32 = jnp.float32

_NC = 2
_NS = 16
_NW = _NC * _NS
_EB = 1024
_IB = 128
_TC_EDGE_BLK = 4096
_NODE_BLK = 2000
_G = 64


def _sc_gather(table, idx2d, n_rows, slow_frac_num=3, slow_frac_den=4):
    d = table.shape[1]
    part = 2560
    unit = _NS * part
    rows_c0 = max(unit, (n_rows * slow_frac_num // slow_frac_den)
                  // unit * unit)
    rows_c1 = n_rows - rows_c0
    mesh = plsc.VectorSubcoreMesh(core_axis_name="c", subcore_axis_name="s")

    max_ops = max(rows_c0, rows_c1) // _NS // _IB

    @functools.partial(
        pl.kernel,
        mesh=mesh,
        out_type=jax.ShapeDtypeStruct((n_rows, d), F32),
        scratch_types=[
            pltpu.VMEM((max_ops, _IB), jnp.int32),
            pltpu.VMEM((part, d), F32),
            pltpu.SemaphoreType.DMA,
        ],
        compiler_params=pltpu.CompilerParams(use_tc_tiling_on_sc=False),
    )
    def k(table_hbm, idx_hbm, out_hbm, idx_v, rows_v, sem):
        cid = lax.axis_index("c")
        sid = lax.axis_index("s")

        def run_core(core_base, rows_per_tile):
            tbase = core_base + sid * rows_per_tile
            n_ops_t = rows_per_tile // _IB
            pltpu.sync_copy(idx_hbm.at[pl.ds(tbase // _IB, n_ops_t)],
                            idx_v.at[pl.ds(0, n_ops_t)])
            for pt in range(rows_per_tile // part):
                copies = []
                for j in range(part // _IB):
                    jj = pt * (part // _IB) + j
                    copies.append(pltpu.async_copy(
                        table_hbm.at[idx_v.at[jj]],
                        rows_v.at[pl.ds(j * _IB, _IB)],
                        sem,
                    ))
                for c in copies:
                    c.wait()
                pltpu.sync_copy(
                    rows_v, out_hbm.at[pl.ds(tbase + pt * part, part)])

        @pl.when(cid == 0)
        def _():
            run_core(0, rows_c0 // _NS)

        @pl.when(cid == 1)
        def _():
            run_core(rows_c0, rows_c1 // _NS)

    return k(table, idx2d)


def _sc_scatter(msg, dst2d, zinit):
    n_rows, d = msg.shape
    n_pad = zinit.shape[0]
    rows_per_w = n_rows // _NW
    chunks = rows_per_w // _EB
    node_rows_per_tile = n_pad // _NS
    mesh = plsc.VectorSubcoreMesh(core_axis_name="c", subcore_axis_name="s")

    n_ops = rows_per_w // _IB
    half = rows_per_w // 2

    @functools.partial(
        pl.kernel,
        mesh=mesh,
        out_type=jax.ShapeDtypeStruct((_NC, n_pad, d), F32),
        scratch_types=[
            pltpu.VMEM((n_ops, _IB), jnp.int32),
            pltpu.VMEM((half, d), F32),
            pltpu.VMEM_SHARED((n_pad, d), F32),
            pltpu.SemaphoreType.DMA,
        ],
        compiler_params=pltpu.CompilerParams(use_tc_tiling_on_sc=False),
    )
    def k(msg_hbm, dst_hbm, z_hbm, out_hbm, idx_v, msg_v, shared, sem):
        cid = lax.axis_index("c")
        sid = lax.axis_index("s")
        wid = sid * _NC + cid
        tb = sid * node_rows_per_tile
        pltpu.sync_copy(z_hbm.at[pl.ds(tb, node_rows_per_tile)],
                        shared.at[pl.ds(tb, node_rows_per_tile)])
        pltpu.sync_copy(dst_hbm.at[pl.ds(wid * n_ops, n_ops)], idx_v)
        plsc.subcore_barrier()
        for hlf in range(2):
            pltpu.sync_copy(msg_hbm.at[pl.ds(wid * rows_per_w + hlf * half,
                                             half)], msg_v)
            copies = []
            for j in range(n_ops // 2):
                jj = hlf * (n_ops // 2) + j
                copies.append(pltpu.async_copy(
                    msg_v.at[pl.ds(j * _IB, _IB)],
                    shared.at[idx_v.at[jj]],
                    sem,
                    add=True,
                ))
            for c in copies:
                c.wait()
        plsc.subcore_barrier()
        pltpu.sync_copy(shared.at[pl.ds(tb, node_rows_per_tile)],
                        out_hbm.at[cid, pl.ds(tb, node_rows_per_tile)])

    return k(msg, dst2d, zinit)


def _tc_msg(xj, ea, wcatb, rmat, ed, h):
    n_rows = xj.shape[0]
    kw = ed * h

    def body(xj_ref, ea_ref, w_ref, r_ref, out_ref):
        p = jnp.dot(xj_ref[...], w_ref[...], preferred_element_type=F32)
        er = jnp.dot(ea_ref[...], r_ref[...], preferred_element_type=F32)
        t = er * p[:, :kw]
        u = t[:, 0:128]
        for i in range(1, kw // 128):
            u = u + t[:, i * 128:(i + 1) * 128]
        m = p[:, kw:]
        for j in range(128 // h):
            m = m + u[:, j * h:(j + 1) * h]
        out_ref[...] = m

    grid = n_rows // _TC_EDGE_BLK
    return pl.pallas_call(
        body,
        grid=(grid,),
        in_specs=[
            pl.BlockSpec((_TC_EDGE_BLK, xj.shape[1]), lambda i: (i, 0)),
            pl.BlockSpec((_TC_EDGE_BLK, ea.shape[1]), lambda i: (i, 0)),
            pl.BlockSpec((wcatb.shape[0], kw + h), lambda i: (0, 0)),
            pl.BlockSpec((ed, kw), lambda i: (0, 0)),
        ],
        out_specs=pl.BlockSpec((_TC_EDGE_BLK, h), lambda i: (i, 0)),
        out_shape=jax.ShapeDtypeStruct((n_rows, h), F32),
    )(xj, ea, wcatb, rmat)


def _tc_combine(p, x, root, bias):
    n, d = x.shape
    h = root.shape[1]

    def body(p_ref, x_ref, r_ref, b_ref, out_ref):
        agg = p_ref[0] + p_ref[1]
        xr = jnp.dot(x_ref[...], r_ref[...], preferred_element_type=F32)
        out_ref[...] = jnp.maximum(agg + xr + b_ref[...], 0.0)

    grid = n // _NODE_BLK
    return pl.pallas_call(
        body,
        grid=(grid,),
        in_specs=[
            pl.BlockSpec((2, _NODE_BLK, h), lambda i: (0, i, 0)),
            pl.BlockSpec((_NODE_BLK, d), lambda i: (i, 0)),
            pl.BlockSpec((d, h), lambda i: (0, 0)),
            pl.BlockSpec((1, h), lambda i: (0, 0)),
        ],
        out_specs=pl.BlockSpec((_NODE_BLK, h), lambda i: (i, 0)),
        out_shape=jax.ShapeDtypeStruct((n, h), F32),
    )(p, x, root, bias)


def _tc_final(p, h1, root2, bias2, batch3, wp1, bp1, wp2, bp2):
    n, h = h1.shape
    out_d = wp2.shape[1]
    grid = n // _NODE_BLK

    def body(p_ref, h1_ref, r2_ref, b2_ref, b_ref, wp1_ref, bp1_ref,
             wp2_ref, bp2_ref, out_ref, pool_acc, cnt_acc):
        i = pl.program_id(0)

        @pl.when(i == 0)
        def _():
            pool_acc[...] = jnp.zeros_like(pool_acc)
            cnt_acc[...] = jnp.zeros_like(cnt_acc)

        hr = jnp.dot(h1_ref[...], r2_ref[...], preferred_element_type=F32)
        h2 = jnp.maximum(p_ref[0] + p_ref[1] + hr + b2_ref[...], 0.0)
        b = b_ref[0, 0, :]
        onehot = (b[:, None] == lax.broadcasted_iota(jnp.int32, (1, _G), 1)
                  ).astype(F32)
        pool_acc[...] += lax.dot_general(
            onehot, h2, (((0,), (0,)), ((), ())), preferred_element_type=F32)
        cnt_acc[...] += lax.dot_general(
            onehot, jnp.ones((_NODE_BLK, 8), F32),
            (((0,), (0,)), ((), ())), preferred_element_type=F32)

        @pl.when(i == pl.num_programs(0) - 1)
        def _():
            cnt = jnp.maximum(cnt_acc[:, 0:1], 1.0)
            pooled = pool_acc[...] / cnt
            o = jnp.dot(pooled, wp1_ref[...],
                        preferred_element_type=F32) + bp1_ref[...]
            out_ref[...] = jnp.dot(o, wp2_ref[...],
                                   preferred_element_type=F32) + bp2_ref[...]

    return pl.pallas_call(
        body,
        grid=(grid,),
        in_specs=[
            pl.BlockSpec((2, _NODE_BLK, h), lambda i: (0, i, 0)),
            pl.BlockSpec((_NODE_BLK, h), lambda i: (i, 0)),
            pl.BlockSpec((h, h), lambda i: (0, 0)),
            pl.BlockSpec((1, h), lambda i: (0, 0)),
            pl.BlockSpec((1, 1, _NODE_BLK), lambda i: (i, 0, 0)),
            pl.BlockSpec((h, h), lambda i: (0, 0)),
            pl.BlockSpec((1, h), lambda i: (0, 0)),
            pl.BlockSpec((h, out_d), lambda i: (0, 0)),
            pl.BlockSpec((1, out_d), lambda i: (0, 0)),
        ],
        out_specs=pl.BlockSpec((_G, out_d), lambda i: (0, 0)),
        out_shape=jax.ShapeDtypeStruct((_G, out_d), F32),
        scratch_shapes=[
            pltpu.VMEM((_G, h), F32),
            pltpu.VMEM((_G, 8), F32),
        ],
    )(p, h1, root2, bias2, batch3, wp1, bp1, wp2, bp2)


def _fold_edge_weight(we, in_ch, ed, h):
    return we.reshape(ed, in_ch, h).transpose(1, 0, 2).reshape(in_ch, ed * h)


def kernel(x, edge_index, edge_attr, batch, W1e, b1e, root1, bias1,
           W2e, b2e, root2, bias2, Wp1, bp1, Wp2, bp2):
    n, in_ch = x.shape
    e = edge_index.shape[1]
    ed = edge_attr.shape[1]
    h = root1.shape[1]

    unit = _NW * _EB
    epad = ((e + unit - 1) // unit) * unit
    pad = epad - e
    src = jnp.concatenate([edge_index[0], jnp.zeros((pad,), jnp.int32)])
    dst = jnp.concatenate([edge_index[1], jnp.full((pad,), n, jnp.int32)])
    ea = jnp.concatenate([edge_attr, jnp.zeros((pad, ed), F32)], axis=0)
    src2d = src.reshape(-1, _IB)
    dst2d = dst.reshape(-1, _IB)

    n_pad = ((n + _IB * _NS) // (_IB * _NS)) * (_IB * _NS)
    zinit = jnp.zeros((n_pad, h), F32)

    kw = ed * h
    wc1 = jnp.concatenate(
        [_fold_edge_weight(W1e, in_ch, ed, h), b1e.reshape(in_ch, h)], axis=1)
    wc2 = jnp.concatenate(
        [_fold_edge_weight(W2e, h, ed, h), b2e.reshape(h, h)], axis=1)
    rmat = (jnp.arange(kw)[None, :] // h == jnp.arange(ed)[:, None]
            ).astype(F32)

    xj = _sc_gather(x, src2d, epad)
    msg1 = _tc_msg(xj, ea, wc1, rmat, ed, h)
    p1 = _sc_scatter(msg1, dst2d, zinit)
    h1 = _tc_combine(p1[:, :n, :], x, root1, bias1.reshape(1, h))

    hj = _sc_gather(h1, src2d, epad)
    msg2 = _tc_msg(hj, ea, wc2, rmat, ed, h)
    p2 = _sc_scatter(msg2, dst2d, zinit)

    batch3 = batch.reshape(n // _NODE_BLK, 1, _NODE_BLK)
    return _tc_final(p2[:, :n, :], h1, root2, bias2.reshape(1, h),
                     batch3, Wp1, bp1.reshape(1, h), Wp2,
                     bp2.reshape(1, Wp2.shape[1]))
